# Initial kernel scaffold; baseline (speedup 1.0000x reference)
import jax
import jax.numpy as jnp
from jax.experimental import pallas as pl


def kernel(edge_index, edge_values, emb_table, alpha):
    raise NotImplementedError("write your pallas kernel here")


# speedup vs baseline: 1.9060x; 1.9060x over previous
import functools

import jax
import jax.numpy as jnp
from jax import lax
from jax.experimental import pallas as pl
from jax.experimental.pallas import tpu as pltpu
from jax.experimental.pallas import tpu_sc as plsc

NC = 2
NS = 16
LANES = 16


def _make_layer(n, d, e, *, layer_idx, first):
    half = n // NC
    CH = 80
    EP = e // NS
    NCHUNK = EP // CH
    RB = 40
    step_rows = NS * RB
    ACC = ((half + 64 + step_rows - 1) // step_rows) * step_rows
    TPB = ACC // NS
    NRB = TPB // RB

    assert e % (NS * CH) == 0 and half % RB == 0 and d % LANES == 0

    mesh = plsc.VectorSubcoreMesh(core_axis_name="c", subcore_axis_name="s")
    sds = jax.ShapeDtypeStruct

    @functools.partial(
        pl.kernel,
        mesh=mesh,
        out_type=(sds((n, d), jnp.float32), sds((n, d), jnp.float32)),
        scratch_types=[
            pltpu.VMEM((CH,), jnp.int32),
            pltpu.VMEM((CH,), jnp.int32),
            pltpu.VMEM((CH,), jnp.int32),
            pltpu.VMEM((CH,), jnp.float32),
            pltpu.VMEM((CH, d), jnp.float32),
            pltpu.VMEM((RB, d), jnp.float32),
            pltpu.VMEM((RB, d), jnp.float32),
            pltpu.VMEM((16,), jnp.float32),
            pltpu.VMEM_SHARED((ACC, d), jnp.float32),
            pltpu.SemaphoreType.DMA,
        ],
    )
    def step(x_hbm, src_hbm, dst_hbm, w_hbm, outp_hbm, alpha_hbm,
             xn_hbm, outn_hbm,
             gidx_v, didx_v, sidx_v, w_v, rows_v, abuf, obuf, alpha_v, acc,
             sem):
        c = lax.axis_index("c")
        s = lax.axis_index("s")
        base_node = c * half

        def _zrow(i, carry):
            for j in range(d // LANES):
                abuf[i, pl.ds(j * LANES, LANES)] = jnp.zeros((LANES,), jnp.float32)
            return carry
        lax.fori_loop(0, RB, _zrow, 0)
        for k in range(NRB):
            pltpu.sync_copy(abuf, acc.at[pl.ds(s * TPB + k * RB, RB)])
        pltpu.sync_copy(alpha_hbm, alpha_v)
        plsc.subcore_barrier()

        lane = lax.broadcasted_iota(jnp.int32, (LANES,), 0)

        def _chunk(ci, carry):
            be = s * EP + ci * CH
            pltpu.sync_copy(src_hbm.at[pl.ds(be, CH)], gidx_v)
            pltpu.sync_copy(dst_hbm.at[pl.ds(be, CH)], didx_v)
            pltpu.sync_copy(w_hbm.at[pl.ds(be, CH)], w_v)
            pltpu.async_copy(x_hbm.at[gidx_v], rows_v, sem).wait()

            def _grp(g, cc):
                dd = didx_v[pl.ds(g * LANES, LANES)]
                loc = dd - base_node
                ok = (loc >= 0) & (loc < half)
                dummy = half + ((g * LANES + lane) & 63)
                sidx_v[pl.ds(g * LANES, LANES)] = jnp.where(ok, loc, dummy)
                return cc
            lax.fori_loop(0, CH // LANES, _grp, 0)

            def _scale(g, cc):
                w16 = w_v[pl.ds(g * LANES, LANES)]
                for k in range(LANES):
                    wv = jnp.full((LANES,), w16[k], jnp.float32)
                    for j in range(d // LANES):
                        sl = pl.ds(j * LANES, LANES)
                        rows_v[g * LANES + k, sl] = rows_v[g * LANES + k, sl] * wv
                return cc
            lax.fori_loop(0, CH // LANES, _scale, 0)

            pltpu.sync_copy(rows_v, acc.at[sidx_v], add=True)
            return carry
        lax.fori_loop(0, NCHUNK, _chunk, 0)
        plsc.subcore_barrier()

        alpha_all = alpha_v[pl.ds(0, LANES)]
        a_new = jnp.full((LANES,), alpha_all[layer_idx], jnp.float32)
        a_zero = jnp.full((LANES,), alpha_all[0], jnp.float32)
        for k in range(NRB):
            r0 = s * TPB + k * RB

            @pl.when(r0 < half)
            def _():
                g0 = base_node + r0
                pltpu.sync_copy(acc.at[pl.ds(r0, RB)], abuf)
                pltpu.sync_copy(abuf, xn_hbm.at[pl.ds(g0, RB)])
                if first:
                    pltpu.sync_copy(x_hbm.at[pl.ds(g0, RB)], obuf)
                else:
                    pltpu.sync_copy(outp_hbm.at[pl.ds(g0, RB)], obuf)

                def _mix(i3, cc):
                    for j in range(d // LANES):
                        sl = pl.ds(j * LANES, LANES)
                        if first:
                            obuf[i3, sl] = a_zero * obuf[i3, sl] + a_new * abuf[i3, sl]
                        else:
                            obuf[i3, sl] = obuf[i3, sl] + a_new * abuf[i3, sl]
                    return cc
                lax.fori_loop(0, RB, _mix, 0)
                pltpu.sync_copy(obuf, outn_hbm.at[pl.ds(g0, RB)])

    return step


def kernel(edge_index, edge_values, emb_table, alpha):
    n, d = emb_table.shape
    e = edge_values.shape[0]
    src = edge_index[1]
    dst = edge_index[0]
    alpha_pad = jnp.zeros((16,), jnp.float32).at[: alpha.shape[0]].set(alpha)

    x = emb_table
    out = x
    for l in range(1, 4):
        step = _make_layer(n, d, e, layer_idx=l, first=(l == 1))
        x, out = step(x, src, dst, edge_values, out, alpha_pad)

    half = n // 2
    return out[:half], out[half:]


# speedup vs baseline: 4.0975x; 2.1498x over previous
import functools

import jax
import jax.numpy as jnp
from jax import lax
from jax.experimental import pallas as pl
from jax.experimental.pallas import tpu as pltpu
from jax.experimental.pallas import tpu_sc as plsc

NC = 2
NS = 16
LANES = 16
CH = 80


def _make_layer(n, d, e, *, layer_idx, first):
    half = n // NC
    EP = e // NS
    NCHUNK = EP // CH
    RB = 40
    step_rows = NS * RB
    ACC = ((half + 64 + step_rows - 1) // step_rows) * step_rows
    TPB = ACC // NS
    NRB = TPB // RB

    assert e % (NS * CH) == 0 and half % RB == 0 and d % LANES == 0
    assert NCHUNK % 2 == 0

    mesh = plsc.VectorSubcoreMesh(core_axis_name="c", subcore_axis_name="s")
    sds = jax.ShapeDtypeStruct

    @functools.partial(
        pl.kernel,
        mesh=mesh,
        out_type=(sds((n, d), jnp.float32), sds((n, d), jnp.float32)),
        scratch_types=[
            pltpu.VMEM((EP,), jnp.int32),
            pltpu.VMEM((EP,), jnp.int32),
            pltpu.VMEM((EP,), jnp.float32),
            pltpu.VMEM((CH,), jnp.int32),
            pltpu.VMEM((CH,), jnp.int32),
            pltpu.VMEM((CH, d), jnp.float32),
            pltpu.VMEM((CH, d), jnp.float32),
            pltpu.VMEM((16,), jnp.float32),
            pltpu.VMEM_SHARED((ACC, d), jnp.float32),
            pltpu.SemaphoreType.DMA,
            pltpu.SemaphoreType.DMA,
            pltpu.SemaphoreType.DMA,
        ],
    )
    def step(x_hbm, src_hbm, dst_hbm, w_hbm, outp_hbm, alpha_hbm,
             xn_hbm, outn_hbm,
             src_all, dst_all, w_all, sidx0, sidx1, rows0, rows1,
             alpha_v, acc, gsem0, gsem1, esem):
        abuf, obuf = rows0, rows1
        c = lax.axis_index("c")
        s = lax.axis_index("s")
        base_node = c * half

        pltpu.async_copy(src_hbm.at[pl.ds(s * EP, EP)], src_all, esem)
        pltpu.async_copy(dst_hbm.at[pl.ds(s * EP, EP)], dst_all, esem)
        pltpu.async_copy(w_hbm.at[pl.ds(s * EP, EP)], w_all, esem)
        pltpu.sync_copy(alpha_hbm, alpha_v)

        def _zrow(i, carry):
            for j in range(d // LANES):
                abuf[i, pl.ds(j * LANES, LANES)] = jnp.zeros((LANES,), jnp.float32)
            return carry
        lax.fori_loop(0, RB, _zrow, 0)
        for k in range(NRB):
            pltpu.sync_copy(abuf.at[pl.ds(0, RB)],
                            acc.at[pl.ds(s * TPB + k * RB, RB)])

        pltpu.make_async_copy(src_hbm.at[pl.ds(s * EP, EP)], src_all, esem).wait()
        pltpu.make_async_copy(dst_hbm.at[pl.ds(s * EP, EP)], dst_all, esem).wait()
        pltpu.make_async_copy(w_hbm.at[pl.ds(s * EP, EP)], w_all, esem).wait()
        plsc.subcore_barrier()

        lane = lax.broadcasted_iota(jnp.int32, (LANES,), 0)
        bufs = ((rows0, sidx0, gsem0), (rows1, sidx1, gsem1))

        def _issue(ci, b):
            rows, _, gsem = bufs[b]
            pltpu.async_copy(x_hbm.at[src_all.at[pl.ds(ci * CH, CH)]], rows, gsem)

        def _process(ci, b):
            rows, sidx, gsem = bufs[b]
            pltpu.make_async_copy(
                x_hbm.at[src_all.at[pl.ds(ci * CH, CH)]], rows, gsem).wait()

            for g in range(CH // LANES):
                dd = dst_all[pl.ds(ci * CH + g * LANES, LANES)]
                loc = dd - base_node
                ok = (loc >= 0) & (loc < half)
                dummy = half + ((g * LANES + lane) & 63)
                sidx[pl.ds(g * LANES, LANES)] = jnp.where(ok, loc, dummy)
                w16 = w_all[pl.ds(ci * CH + g * LANES, LANES)]
                for k in range(LANES):
                    wv = jnp.full((LANES,), w16[k], jnp.float32)
                    for j in range(d // LANES):
                        sl = pl.ds(j * LANES, LANES)
                        rows[g * LANES + k, sl] = rows[g * LANES + k, sl] * wv

            pltpu.sync_copy(rows, acc.at[sidx], add=True)

        _issue(0, 0)
        _issue(1, 1)

        def _pair(co, carry):
            ci = co * 2
            _process(ci, 0)
            _issue(ci + 2, 0)
            _process(ci + 1, 1)
            _issue(ci + 3, 1)
            return carry
        lax.fori_loop(0, NCHUNK // 2 - 1, _pair, 0)
        _process(NCHUNK - 2, 0)
        _process(NCHUNK - 1, 1)

        plsc.subcore_barrier()

        alpha_all = alpha_v[pl.ds(0, LANES)]
        a_new = jnp.full((LANES,), alpha_all[layer_idx], jnp.float32)
        a_zero = jnp.full((LANES,), alpha_all[0], jnp.float32)
        for k in range(NRB):
            r0 = s * TPB + k * RB

            @pl.when(r0 < half)
            def _():
                g0 = base_node + r0
                ab = abuf.at[pl.ds(0, RB)]
                ob = obuf.at[pl.ds(0, RB)]
                pltpu.sync_copy(acc.at[pl.ds(r0, RB)], ab)
                pltpu.async_copy(ab, xn_hbm.at[pl.ds(g0, RB)], esem)
                if first:
                    pltpu.sync_copy(x_hbm.at[pl.ds(g0, RB)], ob)
                else:
                    pltpu.sync_copy(outp_hbm.at[pl.ds(g0, RB)], ob)

                def _mix(i3, cc):
                    for j in range(d // LANES):
                        sl = pl.ds(j * LANES, LANES)
                        if first:
                            obuf[i3, sl] = a_zero * obuf[i3, sl] + a_new * abuf[i3, sl]
                        else:
                            obuf[i3, sl] = obuf[i3, sl] + a_new * abuf[i3, sl]
                    return cc
                lax.fori_loop(0, RB, _mix, 0)
                pltpu.make_async_copy(ab, xn_hbm.at[pl.ds(g0, RB)], esem).wait()
                pltpu.sync_copy(ob, outn_hbm.at[pl.ds(g0, RB)])

    return step


def kernel(edge_index, edge_values, emb_table, alpha):
    n, d = emb_table.shape
    e = edge_values.shape[0]
    src = edge_index[1]
    dst = edge_index[0]
    w = edge_values
    alpha_pad = jnp.zeros((16,), jnp.float32).at[: alpha.shape[0]].set(alpha)

    x = emb_table
    out = x
    for l in range(1, 4):
        step = _make_layer(n, d, e, layer_idx=l, first=(l == 1))
        x, out = step(x, src, dst, w, out, alpha_pad)

    half = n // 2
    return out[:half], out[half:]


# speedup vs baseline: 4.8713x; 1.1888x over previous
import functools

import jax
import jax.numpy as jnp
from jax import lax
from jax.experimental import pallas as pl
from jax.experimental.pallas import tpu as pltpu
from jax.experimental.pallas import tpu_sc as plsc

NC = 2
NS = 16
LANES = 16
CH = 80
NB = 4


def _make_layer(n, d, e, *, layer_idx, first):
    half = n // NC
    EP = e // NS
    NCHUNK = EP // CH
    RB = 40
    step_rows = NS * RB
    ACC = ((half + 64 + step_rows - 1) // step_rows) * step_rows
    TPB = ACC // NS
    NRB = TPB // RB

    assert e % (NS * CH) == 0 and half % RB == 0 and d % LANES == 0
    assert NCHUNK % 2 == 0 and NCHUNK >= 8

    mesh = plsc.VectorSubcoreMesh(core_axis_name="c", subcore_axis_name="s")
    sds = jax.ShapeDtypeStruct

    @functools.partial(
        pl.kernel,
        mesh=mesh,
        out_type=(sds((n, d), jnp.float32), sds((n, d), jnp.float32)),
        scratch_types=[
            pltpu.VMEM((EP,), jnp.int32),
            pltpu.VMEM((EP,), jnp.int32),
            pltpu.VMEM((16,), jnp.float32),
            pltpu.VMEM_SHARED((ACC, d), jnp.float32),
            [pltpu.VMEM((CH, d), jnp.float32) for _ in range(NB)],
            [pltpu.VMEM((CH,), jnp.int32) for _ in range(NB)],
            [pltpu.VMEM((CH,), jnp.float32) for _ in range(NB)],
            [pltpu.SemaphoreType.DMA for _ in range(NB)],
            [pltpu.SemaphoreType.DMA for _ in range(NB)],
            [pltpu.SemaphoreType.DMA for _ in range(NB)],
            pltpu.SemaphoreType.DMA,
        ],
    )
    def step(x_hbm, src_hbm, dst_hbm, w_hbm, outp_hbm, alpha_hbm,
             xn_hbm, outn_hbm,
             src_all, dst_all, alpha_v, acc, rows, sidx, wbuf,
             gsem, ssem, wsem, esem):
        c = lax.axis_index("c")
        s = lax.axis_index("s")
        base_node = c * half
        lane = lax.broadcasted_iota(jnp.int32, (LANES,), 0)

        def _issue_w(ci, b):
            pltpu.async_copy(w_hbm.at[pl.ds(s * EP + ci * CH, CH)],
                             wbuf[b], wsem[b])

        def _wait_w(ci, b):
            pltpu.make_async_copy(w_hbm.at[pl.ds(s * EP + ci * CH, CH)],
                                  wbuf[b], wsem[b]).wait()

        def _issue_g(ci, b):
            pltpu.async_copy(x_hbm.at[src_all.at[pl.ds(ci * CH, CH)]],
                             rows[b], gsem[b])

        def _wait_g(ci, b):
            pltpu.make_async_copy(x_hbm.at[src_all.at[pl.ds(ci * CH, CH)]],
                                  rows[b], gsem[b]).wait()

        def _issue_s(b):
            pltpu.async_copy(rows[b], acc.at[sidx[b]], ssem[b], add=True)

        def _wait_s(b):
            pltpu.make_async_copy(rows[b], acc.at[sidx[b]], ssem[b]).wait()

        def _compute(ci, b):
            for g in range(CH // LANES):
                dd = dst_all[pl.ds(ci * CH + g * LANES, LANES)]
                loc = dd - base_node
                ok = (loc >= 0) & (loc < half)
                dummy = half + ((g * LANES + lane) & 63)
                sidx[b][pl.ds(g * LANES, LANES)] = jnp.where(ok, loc, dummy)
                w16 = wbuf[b][pl.ds(g * LANES, LANES)]
                for k in range(LANES):
                    wv = jnp.full((LANES,), w16[k], jnp.float32)
                    for j in range(d // LANES):
                        sl = pl.ds(j * LANES, LANES)
                        r = rows[b]
                        r[g * LANES + k, sl] = r[g * LANES + k, sl] * wv

        pltpu.sync_copy(src_hbm.at[pl.ds(s * EP, EP)], src_all)
        pltpu.async_copy(dst_hbm.at[pl.ds(s * EP, EP)], dst_all, esem)
        pltpu.sync_copy(alpha_hbm, alpha_v)
        for b in range(NB):
            _issue_w(b, b)
        _issue_g(0, 0)
        _issue_g(1, 1)

        zb = rows[NB - 1]

        def _zrow(i, carry):
            for j in range(d // LANES):
                zb[i, pl.ds(j * LANES, LANES)] = jnp.zeros((LANES,), jnp.float32)
            return carry
        lax.fori_loop(0, RB, _zrow, 0)
        for k in range(NRB):
            pltpu.sync_copy(zb.at[pl.ds(0, RB)],
                            acc.at[pl.ds(s * TPB + k * RB, RB)])
        pltpu.make_async_copy(dst_hbm.at[pl.ds(s * EP, EP)], dst_all, esem).wait()
        plsc.subcore_barrier()

        def _iter(ci, b, *, s_wait, g_issue, w_issue):
            if s_wait:
                _wait_s((b + 2) % NB)
            if g_issue:
                _issue_g(ci + 2, (b + 2) % NB)
            _wait_w(ci, b)
            _wait_g(ci, b)
            _compute(ci, b)
            _issue_s(b)
            if w_issue:
                _issue_w(ci + NB, b)

        _iter(0, 0, s_wait=False, g_issue=True, w_issue=True)
        _iter(1, 1, s_wait=False, g_issue=True, w_issue=True)

        NQ = (NCHUNK - 6) // NB

        def _quad(q, carry):
            ci0 = q * NB + 2
            for o in range(NB):
                _iter(ci0 + o, (2 + o) % NB, s_wait=True, g_issue=True,
                      w_issue=True)
            return carry
        lax.fori_loop(0, NQ, _quad, 0)

        base_t = NQ * NB + 2
        for o in range(4):
            ci = base_t + o
            _iter(ci, (2 + o) % NB, s_wait=True, g_issue=(o < 2),
                  w_issue=False)
        _wait_s((2 + 2) % NB)
        _wait_s((2 + 3) % NB)
        plsc.subcore_barrier()

        abuf, obuf = rows[0], rows[1]
        alpha_all = alpha_v[pl.ds(0, LANES)]
        a_new = jnp.full((LANES,), alpha_all[layer_idx], jnp.float32)
        a_zero = jnp.full((LANES,), alpha_all[0], jnp.float32)
        for k in range(NRB):
            r0 = s * TPB + k * RB

            @pl.when(r0 < half)
            def _():
                g0 = base_node + r0
                ab = abuf.at[pl.ds(0, RB)]
                ob = obuf.at[pl.ds(0, RB)]
                pltpu.sync_copy(acc.at[pl.ds(r0, RB)], ab)
                pltpu.async_copy(ab, xn_hbm.at[pl.ds(g0, RB)], esem)
                if first:
                    pltpu.sync_copy(x_hbm.at[pl.ds(g0, RB)], ob)
                else:
                    pltpu.sync_copy(outp_hbm.at[pl.ds(g0, RB)], ob)

                def _mix(i3, cc):
                    for j in range(d // LANES):
                        sl = pl.ds(j * LANES, LANES)
                        if first:
                            obuf[i3, sl] = a_zero * obuf[i3, sl] + a_new * abuf[i3, sl]
                        else:
                            obuf[i3, sl] = obuf[i3, sl] + a_new * abuf[i3, sl]
                    return cc
                lax.fori_loop(0, RB, _mix, 0)
                pltpu.make_async_copy(ab, xn_hbm.at[pl.ds(g0, RB)], esem).wait()
                pltpu.sync_copy(ob, outn_hbm.at[pl.ds(g0, RB)])

    return step


def kernel(edge_index, edge_values, emb_table, alpha):
    n, d = emb_table.shape
    e = edge_values.shape[0]
    src = edge_index[1]
    dst = edge_index[0]
    w = edge_values
    alpha_pad = jnp.zeros((16,), jnp.float32).at[: alpha.shape[0]].set(alpha)

    x = emb_table
    out = x
    for l in range(1, 4):
        step = _make_layer(n, d, e, layer_idx=l, first=(l == 1))
        x, out = step(x, src, dst, w, out, alpha_pad)

    half = n // 2
    return out[:half], out[half:]


# speedup vs baseline: 8.0873x; 1.6602x over previous
import functools

import jax
import jax.numpy as jnp
from jax import lax
from jax.experimental import pallas as pl
from jax.experimental.pallas import tpu as pltpu
from jax.experimental.pallas import tpu_sc as plsc

NC = 2
NS = 16
LANES = 16
CH = 80
NB = 4


def _make_layer(n, d, e, *, layer_idx, first):
    dh = d // NC
    EP = e // NS
    NCHUNK = EP // CH
    RB = 40
    step_rows = NS * RB
    ACC = ((n + step_rows - 1) // step_rows) * step_rows
    TPB = ACC // NS
    NRB = TPB // RB

    assert e % (NS * CH) == 0 and n % RB == 0 and dh % LANES == 0
    assert NCHUNK >= 8 and (NCHUNK - 6) % NB == 0

    mesh = plsc.VectorSubcoreMesh(core_axis_name="c", subcore_axis_name="s")
    sds = jax.ShapeDtypeStruct

    @functools.partial(
        pl.kernel,
        mesh=mesh,
        compiler_params=pltpu.CompilerParams(use_tc_tiling_on_sc=False),
        out_type=(sds((NC, n, dh), jnp.float32), sds((NC, n, dh), jnp.float32)),
        scratch_types=[
            pltpu.VMEM((EP,), jnp.int32),
            pltpu.VMEM((EP,), jnp.int32),
            pltpu.VMEM((16,), jnp.float32),
            pltpu.VMEM_SHARED((ACC, dh), jnp.float32),
            [pltpu.VMEM((CH, dh), jnp.float32) for _ in range(NB)],
            [pltpu.VMEM((CH,), jnp.int32) for _ in range(NB)],
            [pltpu.VMEM((CH,), jnp.float32) for _ in range(NB)],
            [pltpu.SemaphoreType.DMA for _ in range(NB)],
            [pltpu.SemaphoreType.DMA for _ in range(NB)],
            [pltpu.SemaphoreType.DMA for _ in range(NB)],
            pltpu.SemaphoreType.DMA,
        ],
    )
    def step(xs_hbm, src_hbm, dst_hbm, w_hbm, outp_hbm, alpha_hbm,
             xn_hbm, outn_hbm,
             src_all, dst_all, alpha_v, acc, rows, sidx, wbuf,
             gsem, ssem, wsem, esem):
        c = lax.axis_index("c")
        s = lax.axis_index("s")

        def _issue_w(ci, b):
            pltpu.async_copy(w_hbm.at[pl.ds(s * EP + ci * CH, CH)],
                             wbuf[b], wsem[b])

        def _wait_w(ci, b):
            pltpu.make_async_copy(w_hbm.at[pl.ds(s * EP + ci * CH, CH)],
                                  wbuf[b], wsem[b]).wait()

        def _issue_g(ci, b):
            pltpu.async_copy(
                xs_hbm.at[c].at[src_all.at[pl.ds(ci * CH, CH)]],
                rows[b], gsem[b])

        def _wait_g(ci, b):
            pltpu.make_async_copy(
                xs_hbm.at[c].at[src_all.at[pl.ds(ci * CH, CH)]],
                rows[b], gsem[b]).wait()

        def _issue_s(b):
            pltpu.async_copy(rows[b], acc.at[sidx[b]], ssem[b], add=True)

        def _wait_s(b):
            pltpu.make_async_copy(rows[b], acc.at[sidx[b]], ssem[b]).wait()

        def _compute(ci, b):
            for g in range(CH // LANES):
                sidx[b][pl.ds(g * LANES, LANES)] = (
                    dst_all[pl.ds(ci * CH + g * LANES, LANES)])
                w16 = wbuf[b][pl.ds(g * LANES, LANES)]
                for k in range(LANES):
                    wv = jnp.full((LANES,), w16[k], jnp.float32)
                    for j in range(dh // LANES):
                        sl = pl.ds(j * LANES, LANES)
                        r = rows[b]
                        r[g * LANES + k, sl] = r[g * LANES + k, sl] * wv

        pltpu.sync_copy(src_hbm.at[pl.ds(s * EP, EP)], src_all)
        pltpu.async_copy(dst_hbm.at[pl.ds(s * EP, EP)], dst_all, esem)
        pltpu.sync_copy(alpha_hbm, alpha_v)
        for b in range(NB):
            _issue_w(b, b)
        _issue_g(0, 0)
        _issue_g(1, 1)

        zb = rows[NB - 1]

        def _zrow(i, carry):
            for j in range(dh // LANES):
                zb[i, pl.ds(j * LANES, LANES)] = jnp.zeros((LANES,), jnp.float32)
            return carry
        lax.fori_loop(0, RB, _zrow, 0)
        for k in range(NRB):
            pltpu.sync_copy(zb.at[pl.ds(0, RB)],
                            acc.at[pl.ds(s * TPB + k * RB, RB)])
        pltpu.make_async_copy(dst_hbm.at[pl.ds(s * EP, EP)], dst_all, esem).wait()
        plsc.subcore_barrier()

        def _iter(ci, b, *, s_wait, g_issue, w_issue):
            if s_wait:
                _wait_s((b + 2) % NB)
            if g_issue:
                _issue_g(ci + 2, (b + 2) % NB)
            _wait_w(ci, b)
            _wait_g(ci, b)
            _compute(ci, b)
            _issue_s(b)
            if w_issue:
                _issue_w(ci + NB, b)

        _iter(0, 0, s_wait=False, g_issue=True, w_issue=True)
        _iter(1, 1, s_wait=False, g_issue=True, w_issue=True)

        NQ = (NCHUNK - 6) // NB

        def _quad(q, carry):
            ci0 = q * NB + 2
            for o in range(NB):
                _iter(ci0 + o, (2 + o) % NB, s_wait=True, g_issue=True,
                      w_issue=True)
            return carry
        lax.fori_loop(0, NQ, _quad, 0)

        base_t = NQ * NB + 2
        for o in range(4):
            ci = base_t + o
            _iter(ci, (2 + o) % NB, s_wait=True, g_issue=(o < 2),
                  w_issue=False)
        _wait_s(0)
        _wait_s(1)
        plsc.subcore_barrier()

        abuf, obuf = rows[0], rows[1]
        alpha_all = alpha_v[pl.ds(0, LANES)]
        a_new = jnp.full((LANES,), alpha_all[layer_idx], jnp.float32)
        a_zero = jnp.full((LANES,), alpha_all[0], jnp.float32)
        for k in range(NRB):
            r0 = s * TPB + k * RB

            @pl.when(r0 < n)
            def _():
                ab = abuf.at[pl.ds(0, RB)]
                ob = obuf.at[pl.ds(0, RB)]
                pltpu.sync_copy(acc.at[pl.ds(r0, RB)], ab)
                pltpu.async_copy(ab, xn_hbm.at[c].at[pl.ds(r0, RB)], esem)
                if first:
                    pltpu.sync_copy(xs_hbm.at[c].at[pl.ds(r0, RB)], ob)
                else:
                    pltpu.sync_copy(outp_hbm.at[c].at[pl.ds(r0, RB)], ob)

                def _mix(i3, cc):
                    for j in range(dh // LANES):
                        sl = pl.ds(j * LANES, LANES)
                        if first:
                            obuf[i3, sl] = a_zero * obuf[i3, sl] + a_new * abuf[i3, sl]
                        else:
                            obuf[i3, sl] = obuf[i3, sl] + a_new * abuf[i3, sl]
                    return cc
                lax.fori_loop(0, RB, _mix, 0)
                pltpu.make_async_copy(ab, xn_hbm.at[c].at[pl.ds(r0, RB)],
                                     esem).wait()
                pltpu.sync_copy(ob, outn_hbm.at[c].at[pl.ds(r0, RB)])

    return step


def kernel(edge_index, edge_values, emb_table, alpha):
    n, d = emb_table.shape
    e = edge_values.shape[0]
    src = edge_index[1]
    dst = edge_index[0]
    w = edge_values
    dh = d // NC
    alpha_pad = jnp.zeros((16,), jnp.float32).at[: alpha.shape[0]].set(alpha)

    x = jnp.stack([emb_table[:, i * dh:(i + 1) * dh] for i in range(NC)])
    out = x
    for l in range(1, 4):
        step = _make_layer(n, d, e, layer_idx=l, first=(l == 1))
        x, out = step(x, src, dst, w, out, alpha_pad)

    out_full = jnp.concatenate([out[i] for i in range(NC)], axis=1)
    half = n // 2
    return out_full[:half], out_full[half:]


# speedup vs baseline: 8.1526x; 1.0081x over previous
import functools

import jax
import jax.numpy as jnp
from jax import lax
from jax.experimental import pallas as pl
from jax.experimental.pallas import tpu as pltpu
from jax.experimental.pallas import tpu_sc as plsc

NC = 2
NS = 16
LANES = 16
CH = 80
NB = 4
NL = 3


def _make_kernel(n, d, e):
    dh = d // NC
    EP = e // NS
    NCHUNK = EP // CH
    RB = 40
    step_rows = NS * RB
    ACC = ((n + step_rows - 1) // step_rows) * step_rows
    TPB = ACC // NS
    NRB = TPB // RB

    assert e % (NS * CH) == 0 and n % RB == 0 and dh % LANES == 0
    assert NCHUNK >= 8 and (NCHUNK - 6) % NB == 0

    mesh = plsc.VectorSubcoreMesh(core_axis_name="c", subcore_axis_name="s")
    sds = jax.ShapeDtypeStruct

    @functools.partial(
        pl.kernel,
        mesh=mesh,
        compiler_params=pltpu.CompilerParams(use_tc_tiling_on_sc=False),
        out_type=(sds((NC, n, dh), jnp.float32),
                  sds((2, NC, n, dh), jnp.float32)),
        scratch_types=[
            pltpu.VMEM((EP,), jnp.int32),
            pltpu.VMEM((EP,), jnp.int32),
            pltpu.VMEM((16,), jnp.float32),
            pltpu.VMEM((RB, dh), jnp.float32),
            pltpu.VMEM_SHARED((ACC, dh), jnp.float32),
            [pltpu.VMEM((CH, dh), jnp.float32) for _ in range(NB)],
            [pltpu.VMEM((CH,), jnp.int32) for _ in range(NB)],
            [pltpu.VMEM((CH,), jnp.float32) for _ in range(NB)],
            [pltpu.SemaphoreType.DMA for _ in range(NB)],
            [pltpu.SemaphoreType.DMA for _ in range(NB)],
            [pltpu.SemaphoreType.DMA for _ in range(NB)],
            pltpu.SemaphoreType.DMA,
        ],
    )
    def step(xs_hbm, src_hbm, dst_hbm, w_hbm, alpha_hbm,
             out_hbm, xb_hbm,
             src_all, dst_all, alpha_v, zbuf, acc, rows, sidx, wbuf,
             gsem, ssem, wsem, esem):
        c = lax.axis_index("c")
        s = lax.axis_index("s")

        def _issue_w(ci, b):
            pltpu.async_copy(w_hbm.at[pl.ds(s * EP + ci * CH, CH)],
                             wbuf[b], wsem[b])

        def _wait_w(ci, b):
            pltpu.make_async_copy(w_hbm.at[pl.ds(s * EP + ci * CH, CH)],
                                  wbuf[b], wsem[b]).wait()

        def _issue_g(sp, ci, b):
            pltpu.async_copy(
                xb_hbm.at[sp, c].at[src_all.at[pl.ds(ci * CH, CH)]],
                rows[b], gsem[b])

        def _wait_g(sp, ci, b):
            pltpu.make_async_copy(
                xb_hbm.at[sp, c].at[src_all.at[pl.ds(ci * CH, CH)]],
                rows[b], gsem[b]).wait()

        def _issue_s(b):
            pltpu.async_copy(rows[b], acc.at[sidx[b]], ssem[b], add=True)

        def _wait_s(b):
            pltpu.make_async_copy(rows[b], acc.at[sidx[b]], ssem[b]).wait()

        def _compute(ci, b):
            for g in range(CH // LANES):
                sidx[b][pl.ds(g * LANES, LANES)] = (
                    dst_all[pl.ds(ci * CH + g * LANES, LANES)])
                w16 = wbuf[b][pl.ds(g * LANES, LANES)]
                for k in range(LANES):
                    wv = jnp.full((LANES,), w16[k], jnp.float32)
                    for j in range(dh // LANES):
                        sl = pl.ds(j * LANES, LANES)
                        r = rows[b]
                        r[g * LANES + k, sl] = r[g * LANES + k, sl] * wv

        pltpu.sync_copy(src_hbm.at[pl.ds(s * EP, EP)], src_all)
        pltpu.async_copy(dst_hbm.at[pl.ds(s * EP, EP)], dst_all, esem)
        pltpu.sync_copy(alpha_hbm, alpha_v)

        def _zrow(i, carry):
            for j in range(dh // LANES):
                zbuf[i, pl.ds(j * LANES, LANES)] = jnp.zeros((LANES,), jnp.float32)
            return carry
        lax.fori_loop(0, RB, _zrow, 0)

        alpha_all = alpha_v[pl.ds(0, LANES)]
        abuf, obuf = rows[0], rows[1]
        a0 = jnp.full((LANES,), alpha_all[0], jnp.float32)

        for k in range(NRB):
            r0 = s * TPB + k * RB

            @pl.when(r0 < n)
            def _():
                ab = abuf.at[pl.ds(0, RB)]
                ob = obuf.at[pl.ds(0, RB)]
                pltpu.sync_copy(xs_hbm.at[c].at[pl.ds(r0, RB)], ab)
                pltpu.async_copy(ab, xb_hbm.at[0, c].at[pl.ds(r0, RB)], esem)

                def _mix0(i3, cc):
                    for j in range(dh // LANES):
                        sl = pl.ds(j * LANES, LANES)
                        obuf[i3, sl] = a0 * abuf[i3, sl]
                    return cc
                lax.fori_loop(0, RB, _mix0, 0)
                pltpu.sync_copy(ob, out_hbm.at[c].at[pl.ds(r0, RB)])
                pltpu.make_async_copy(ab, xb_hbm.at[0, c].at[pl.ds(r0, RB)],
                                     esem).wait()
            pltpu.sync_copy(zbuf, acc.at[pl.ds(s * TPB + k * RB, RB)])
        pltpu.make_async_copy(dst_hbm.at[pl.ds(s * EP, EP)], dst_all, esem).wait()
        plsc.subcore_barrier()

        def _layer(l, carry):
            sp = (l + 1) % 2
            for b in range(NB):
                _issue_w(b, b)
            _issue_g(sp, 0, 0)
            _issue_g(sp, 1, 1)

            def _iter(ci, b, *, s_wait, g_issue, w_issue):
                if s_wait:
                    _wait_s((b + 2) % NB)
                if g_issue:
                    _issue_g(sp, ci + 2, (b + 2) % NB)
                _wait_w(ci, b)
                _wait_g(sp, ci, b)
                _compute(ci, b)
                _issue_s(b)
                if w_issue:
                    _issue_w(ci + NB, b)

            _iter(0, 0, s_wait=False, g_issue=True, w_issue=True)
            _iter(1, 1, s_wait=False, g_issue=True, w_issue=True)

            NQ = (NCHUNK - 6) // NB

            def _quad(q, cc):
                ci0 = q * NB + 2
                for o in range(NB):
                    _iter(ci0 + o, (2 + o) % NB, s_wait=True, g_issue=True,
                          w_issue=True)
                return cc
            lax.fori_loop(0, NQ, _quad, 0)

            base_t = NQ * NB + 2
            for o in range(4):
                ci = base_t + o
                _iter(ci, (2 + o) % NB, s_wait=True, g_issue=(o < 2),
                      w_issue=False)
            _wait_s(0)
            _wait_s(1)
            plsc.subcore_barrier()

            a1 = jnp.full((LANES,), alpha_all[1], jnp.float32)
            a2 = jnp.full((LANES,), alpha_all[2], jnp.float32)
            a3 = jnp.full((LANES,), alpha_all[3], jnp.float32)
            lv = jnp.full((LANES,), l, jnp.int32)
            a_new = jnp.where(lv == 1, a1, jnp.where(lv == 2, a2, a3))
            for k in range(NRB):
                r0 = s * TPB + k * RB

                @pl.when(r0 < n)
                def _():
                    ab = abuf.at[pl.ds(0, RB)]
                    ob = obuf.at[pl.ds(0, RB)]
                    pltpu.sync_copy(acc.at[pl.ds(r0, RB)], ab)
                    pltpu.async_copy(ab, xb_hbm.at[l % 2, c].at[pl.ds(r0, RB)],
                                     esem)
                    pltpu.sync_copy(out_hbm.at[c].at[pl.ds(r0, RB)], ob)

                    def _mix(i3, cc):
                        for j in range(dh // LANES):
                            sl = pl.ds(j * LANES, LANES)
                            obuf[i3, sl] = obuf[i3, sl] + a_new * abuf[i3, sl]
                        return cc
                    lax.fori_loop(0, RB, _mix, 0)
                    pltpu.sync_copy(ob, out_hbm.at[c].at[pl.ds(r0, RB)])
                    pltpu.make_async_copy(
                        ab, xb_hbm.at[l % 2, c].at[pl.ds(r0, RB)], esem).wait()
                pltpu.sync_copy(zbuf, acc.at[pl.ds(s * TPB + k * RB, RB)])
            plsc.subcore_barrier()
            return carry
        lax.fori_loop(1, NL + 1, _layer, 0)

    return step


def kernel(edge_index, edge_values, emb_table, alpha):
    n, d = emb_table.shape
    e = edge_values.shape[0]
    src = edge_index[1]
    dst = edge_index[0]
    w = edge_values
    dh = d // NC
    alpha_pad = jnp.zeros((16,), jnp.float32).at[: alpha.shape[0]].set(alpha)

    x = jnp.stack([emb_table[:, i * dh:(i + 1) * dh] for i in range(NC)])
    out, _ = _make_kernel(n, d, e)(x, src, dst, w, alpha_pad)

    out_full = jnp.concatenate([out[i] for i in range(NC)], axis=1)
    half = n // 2
    return out_full[:half], out_full[half:]
